# baseline (speedup 1.0000x reference)
import functools

import jax
import jax.numpy as jnp
from jax.experimental import pallas as pl
from jax.experimental.pallas import tpu as pltpu

K = 16


def _knn_block_kernel(x_ref, xyzt_ref, sq_ref, xyz_ref, feat_ref, idx_ref, *, rows, n):
    x = x_ref[:, :]
    sq_r = jnp.sum(x * x, axis=1, keepdims=True)
    cross = jnp.dot(x, xyzt_ref[:, :], preferred_element_type=jnp.float32)
    d = sq_r + sq_ref[0:1, :] - 2.0 * cross

    iota = jax.lax.broadcasted_iota(jnp.int32, (rows, n), 1).astype(jnp.float32)
    feats = []
    idxs = []
    for _ in range(K):
        m = jnp.min(d, axis=1, keepdims=True)
        idxf = jnp.min(jnp.where(d == m, iota, jnp.inf), axis=1, keepdims=True)
        onehot = iota == idxf
        nb = jnp.dot(jnp.where(onehot, 1.0, 0.0), xyz_ref[:, :],
                     preferred_element_type=jnp.float32)
        d = jnp.where(onehot, jnp.inf, d)
        feats.append(jnp.concatenate([m, x - nb, x, nb], axis=1))
        idxs.append(idxf)
    feat_ref[:, :] = jnp.concatenate(feats, axis=1)
    idx_ref[:, :] = jnp.concatenate(idxs, axis=1).astype(jnp.int32)


def _knn_features(pts, rows):
    n = pts.shape[0]
    sq = jnp.sum(pts * pts, axis=-1)[None, :]
    grid = (n // rows,)
    feat, idx = pl.pallas_call(
        functools.partial(_knn_block_kernel, rows=rows, n=n),
        grid=grid,
        in_specs=[
            pl.BlockSpec((rows, 3), lambda i: (i, 0)),
            pl.BlockSpec((3, n), lambda i: (0, 0)),
            pl.BlockSpec((1, n), lambda i: (0, 0)),
            pl.BlockSpec((n, 3), lambda i: (0, 0)),
        ],
        out_specs=[
            pl.BlockSpec((rows, K * 10), lambda i: (i, 0)),
            pl.BlockSpec((rows, K), lambda i: (i, 0)),
        ],
        out_shape=[
            jax.ShapeDtypeStruct((n, K * 10), jnp.float32),
            jax.ShapeDtypeStruct((n, K), jnp.int32),
        ],
        compiler_params=pltpu.CompilerParams(
            dimension_semantics=("parallel",)),
    )(pts, pts.T, sq, pts)
    return feat, idx


def kernel(xyz):
    b, n, _ = xyz.shape
    feat, idx = jax.vmap(lambda p: _knn_features(p, 64))(xyz.reshape(b, n, 3))
    return feat.reshape(b, n, K, 10), idx.reshape(b, n, K)


# speedup vs baseline: 1.0373x; 1.0373x over previous
import functools

import jax
import jax.numpy as jnp
from jax.experimental import pallas as pl
from jax.experimental.pallas import tpu as pltpu

K = 16


def _knn_block_kernel(x_ref, xyzt_ref, sq_ref, xyz_ref, feat_ref, idx_ref, *, rows, n):
    x = x_ref[:, :]
    sq_r = jnp.sum(x * x, axis=1, keepdims=True)
    cross = jnp.dot(x, xyzt_ref[:, :], preferred_element_type=jnp.float32)
    d = sq_r + sq_ref[0:1, :] - 2.0 * cross

    iota = jax.lax.broadcasted_iota(jnp.int32, (rows, n), 1).astype(jnp.float32)
    feats = []
    idxs = []
    for _ in range(K):
        m = jnp.min(d, axis=1, keepdims=True)
        idxf = jnp.min(jnp.where(d == m, iota, jnp.inf), axis=1, keepdims=True)
        onehot = iota == idxf
        nb = jnp.dot(jnp.where(onehot, 1.0, 0.0), xyz_ref[:, :],
                     preferred_element_type=jnp.float32)
        d = jnp.where(onehot, jnp.inf, d)
        feats.append(jnp.concatenate([m, x - nb, x, nb], axis=1))
        idxs.append(idxf)
    feat_ref[:, :] = jnp.concatenate(feats, axis=1)
    idx_ref[:, :] = jnp.concatenate(idxs, axis=1).astype(jnp.int32)


def _knn_features(pts, rows):
    n = pts.shape[0]
    sq = jnp.sum(pts * pts, axis=-1)[None, :]
    grid = (n // rows,)
    feat, idx = pl.pallas_call(
        functools.partial(_knn_block_kernel, rows=rows, n=n),
        grid=grid,
        in_specs=[
            pl.BlockSpec((rows, 3), lambda i: (i, 0)),
            pl.BlockSpec((3, n), lambda i: (0, 0)),
            pl.BlockSpec((1, n), lambda i: (0, 0)),
            pl.BlockSpec((n, 3), lambda i: (0, 0)),
        ],
        out_specs=[
            pl.BlockSpec((rows, K * 10), lambda i: (i, 0)),
            pl.BlockSpec((rows, K), lambda i: (i, 0)),
        ],
        out_shape=[
            jax.ShapeDtypeStruct((n, K * 10), jnp.float32),
            jax.ShapeDtypeStruct((n, K), jnp.int32),
        ],
        compiler_params=pltpu.CompilerParams(
            dimension_semantics=("parallel",)),
    )(pts, pts.T, sq, pts)
    return feat, idx


def kernel(xyz):
    b, n, _ = xyz.shape
    feat, idx = jax.vmap(lambda p: _knn_features(p, 128))(xyz.reshape(b, n, 3))
    return feat.reshape(b, n, K, 10), idx.reshape(b, n, K)
